# Initial kernel scaffold; baseline (speedup 1.0000x reference)
import jax
import jax.numpy as jnp
from jax.experimental import pallas as pl


def kernel(X, table, W):
    raise NotImplementedError("write your pallas kernel here")


# speedup vs baseline: 7.6519x; 7.6519x over previous
import functools

import jax
import jax.numpy as jnp
from jax import lax
from jax.experimental import pallas as pl
from jax.experimental.pallas import tpu as pltpu
from jax.experimental.pallas import tpu_sc as plsc

CHAR_SIZE = 100000
EMB_DIM = 30
PROJ_DIM = 300
BATCH = 16384
SEQ = 200

PAD_D = 32
GROUP = PROJ_DIM // EMB_DIM
NIDX = BATCH * SEQ
ROWS = NIDX // GROUP

NC, NS = 2, 16
NW = NC * NS
PER_W = NIDX // NW
RPG = 128
K = 16
CHUNK = K * RPG
ITERS = PER_W // CHUNK
BLOCKS_PER_W = PER_W // RPG

RBLK = 1024


def _sc_gather_body(idx_hbm, table_hbm, out_hbm, idx_v, rows_v, sem):
    wid = lax.axis_index("s") * NC + lax.axis_index("c")

    def outer(i, carry):
        blk0 = wid * BLOCKS_PER_W + i * K
        row0 = wid * PER_W + i * CHUNK
        pltpu.sync_copy(idx_hbm.at[pl.ds(blk0, K)], idx_v)
        cps = [
            pltpu.async_copy(
                table_hbm.at[idx_v.at[j]], rows_v.at[pl.ds(j * RPG, RPG)], sem
            )
            for j in range(K)
        ]
        for cp in cps:
            cp.wait()
        pltpu.sync_copy(rows_v, out_hbm.at[pl.ds(row0, CHUNK)])
        return carry

    lax.fori_loop(0, ITERS, outer, 0)


@functools.lru_cache(maxsize=None)
def _sc_gather():
    return pl.kernel(
        _sc_gather_body,
        out_type=jax.ShapeDtypeStruct((NIDX, PAD_D), jnp.float32),
        mesh=plsc.VectorSubcoreMesh(
            core_axis_name="c", subcore_axis_name="s", num_cores=NC, num_subcores=NS
        ),
        scratch_types=[
            pltpu.VMEM((K, RPG), jnp.int32),
            pltpu.VMEM((CHUNK, PAD_D), jnp.float32),
            pltpu.SemaphoreType.DMA,
        ],
        compiler_params=pltpu.CompilerParams(use_tc_tiling_on_sc=False),
    )


def _mm_body(a_ref, w_ref, o_ref):
    o_ref[...] = jnp.dot(
        a_ref[...], w_ref[...], preferred_element_type=jnp.float32
    )


def _project(a, wp):
    return pl.pallas_call(
        _mm_body,
        grid=(ROWS // RBLK,),
        in_specs=[
            pl.BlockSpec((RBLK, GROUP * PAD_D), lambda i: (i, 0)),
            pl.BlockSpec((GROUP * PAD_D, PROJ_DIM), lambda i: (0, 0)),
        ],
        out_specs=pl.BlockSpec((RBLK, PROJ_DIM), lambda i: (i, 0)),
        out_shape=jax.ShapeDtypeStruct((ROWS, PROJ_DIM), jnp.float32),
    )(a, wp)


def kernel(X, table, W):
    idx = X.reshape(NIDX // RPG, RPG).astype(jnp.int32)
    table_pad = jnp.pad(table, ((0, 0), (0, PAD_D - EMB_DIM)))
    gathered = _sc_gather()(idx, table_pad)
    a = gathered.reshape(ROWS, GROUP * PAD_D)
    wp = jnp.pad(
        W.T.reshape(GROUP, EMB_DIM, PROJ_DIM),
        ((0, 0), (0, PAD_D - EMB_DIM), (0, 0)),
    ).reshape(GROUP * PAD_D, PROJ_DIM)
    return _project(a, wp)


# speedup vs baseline: 9.9657x; 1.3024x over previous
import functools

import jax
import jax.numpy as jnp
from jax import lax
from jax.experimental import pallas as pl
from jax.experimental.pallas import tpu as pltpu
from jax.experimental.pallas import tpu_sc as plsc

CHAR_SIZE = 100000
EMB_DIM = 30
PROJ_DIM = 300
BATCH = 16384
SEQ = 200

PAD_D = 32
GROUP = PROJ_DIM // EMB_DIM
NIDX = BATCH * SEQ
ROWS = NIDX // GROUP

KPAD = 384
NTROW = ROWS // 8
NLINES = ROWS * KPAD // PAD_D
N128 = NLINES // 4

NC, NS = 2, 16
NW = NC * NS
PER_W = NIDX // NW
RPG = 128
K = 16
CHUNK = K * RPG
ITERS = PER_W // CHUNK
BLOCKS_PER_W = PER_W // RPG

RB8 = 128


def _sc_gather_body(idx_hbm, lidx_hbm, table_hbm, out_hbm, idx_v, lidx_v,
                    rows_v, gsem, ssem):
    wid = lax.axis_index("s") * NC + lax.axis_index("c")
    out_lines = out_hbm

    def outer(i, carry):
        blk0 = wid * BLOCKS_PER_W + i * K
        pltpu.sync_copy(idx_hbm.at[pl.ds(blk0, K)], idx_v)
        pltpu.sync_copy(lidx_hbm.at[pl.ds(blk0, K)], lidx_v)
        gcps = [
            pltpu.async_copy(
                table_hbm.at[idx_v.at[j]], rows_v.at[pl.ds(j * RPG, RPG)], gsem
            )
            for j in range(K)
        ]
        for cp in gcps:
            cp.wait()
        scps = [
            pltpu.async_copy(
                rows_v.at[pl.ds(j * RPG, RPG)], out_lines.at[lidx_v.at[j]], ssem
            )
            for j in range(K)
        ]
        for cp in scps:
            cp.wait()
        return carry

    lax.fori_loop(0, ITERS, outer, 0)


@functools.lru_cache(maxsize=None)
def _sc_gather():
    return pl.kernel(
        _sc_gather_body,
        out_type=jax.ShapeDtypeStruct((NLINES, PAD_D), jnp.float32),
        mesh=plsc.VectorSubcoreMesh(
            core_axis_name="c", subcore_axis_name="s", num_cores=NC, num_subcores=NS
        ),
        scratch_types=[
            pltpu.VMEM((K, RPG), jnp.int32),
            pltpu.VMEM((K, RPG), jnp.int32),
            pltpu.VMEM((CHUNK, PAD_D), jnp.float32),
            pltpu.SemaphoreType.DMA,
            pltpu.SemaphoreType.DMA,
        ],
        compiler_params=pltpu.CompilerParams(use_tc_tiling_on_sc=False),
    )


def _dest_lines():
    m = jnp.arange(NIDX, dtype=jnp.int32)
    r = m // GROUP
    j = m - r * GROUP
    return (r // 8) * 96 + (j // 4) * 32 + (r % 8) * 4 + (j % 4)


def _mm_body(a_ref, w_ref, o_ref):
    a4 = a_ref[...].reshape(RB8, 3, 8, 128)
    acc = None
    for c in range(3):
        ac = a4[:, c].reshape(RB8 * 8, 128)
        if c == 2:
            lanes = lax.broadcasted_iota(jnp.int32, (RB8 * 8, 128), 1)
            ac = jnp.where(lanes < 64, ac, 0.0)
        p = jnp.dot(
            ac,
            w_ref[pl.ds(c * 128, 128), :],
            preferred_element_type=jnp.float32,
        )
        acc = p if acc is None else acc + p
    o_ref[...] = acc


def _project(a, w384):
    return pl.pallas_call(
        _mm_body,
        grid=(NTROW // RB8,),
        in_specs=[
            pl.BlockSpec((RB8 * 24, 128), lambda i: (i, 0)),
            pl.BlockSpec((KPAD, PROJ_DIM), lambda i: (0, 0)),
        ],
        out_specs=pl.BlockSpec((RB8 * 8, PROJ_DIM), lambda i: (i, 0)),
        out_shape=jax.ShapeDtypeStruct((ROWS, PROJ_DIM), jnp.float32),
    )(a, w384)


def kernel(X, table, W):
    idx = X.reshape(NIDX // RPG, RPG).astype(jnp.int32)
    lidx = _dest_lines().reshape(NIDX // RPG, RPG)
    table_pad = jnp.pad(table, ((0, 0), (0, PAD_D - EMB_DIM)))
    lines = _sc_gather()(idx, lidx, table_pad)
    packed = lines.reshape(N128, 128)
    wp = jnp.pad(
        W.T.reshape(GROUP, EMB_DIM, PROJ_DIM),
        ((0, 0), (0, PAD_D - EMB_DIM), (0, 0)),
    ).reshape(GROUP * PAD_D, PROJ_DIM)
    w384 = jnp.pad(wp, ((0, KPAD - GROUP * PAD_D), (0, 0)))
    return _project(packed, w384)
